# baseline (speedup 1.0000x reference)
import functools

import jax
import jax.numpy as jnp
from jax import lax
from jax.experimental import pallas as pl
from jax.experimental.pallas import tpu as pltpu
from jax.experimental.pallas import tpu_sc as plsc

N = 10000
E = 320000
D = 128
DL = 16
NID = 2000

NPAD = 10240
NW = 32
CHUNK = 128
NCH = 80
EPT = NCH * CHUNK
EPAD = NW * EPT
RPT = NPAD // 16
NIDPAD = 2048
IDS_PT = NIDPAD // NW

_sc_mesh = plsc.VectorSubcoreMesh(core_axis_name="c", subcore_axis_name="s")
_sc_params = pltpu.CompilerParams(use_tc_tiling_on_sc=False)


def _make_prop_simple(width, a_chunks, b_chunks):
  bmax = max(a_chunks, b_chunks)

  @functools.partial(
      pl.kernel,
      out_type=jax.ShapeDtypeStruct((2, NPAD, width), jnp.float32),
      mesh=_sc_mesh,
      compiler_params=_sc_params,
      scratch_types=[
          pltpu.VMEM((bmax, 2, CHUNK), jnp.int32),
          pltpu.VMEM((CHUNK, width), jnp.float32),
          pltpu.VMEM_SHARED((NPAD, width), jnp.float32),
          pltpu.SemaphoreType.DMA,
      ],
  )
  def prop(xp_hbm, edges_hbm, zeros_hbm, out_hbm, idx, rows, shared, sem):
    c = lax.axis_index("c")
    s = lax.axis_index("s")
    r0 = pl.multiple_of(s * RPT, 8)
    base = jnp.where(c == 0, s * a_chunks, 16 * a_chunks + s * b_chunks)
    nch_my = jnp.where(c == 0, a_chunks, b_chunks)
    base = jnp.minimum(base, 2 * 16 * NCH - bmax)
    pltpu.sync_copy(zeros_hbm.at[pl.ds(r0, RPT)], shared.at[pl.ds(r0, RPT)])
    pltpu.sync_copy(edges_hbm.at[pl.ds(base, bmax)], idx)
    off = jnp.where(c == 0, s * a_chunks - base, 16 * a_chunks + s * b_chunks - base)
    plsc.subcore_barrier()

    def body(j, carry):
      pltpu.async_copy(xp_hbm.at[idx.at[off + j, 0]], rows, sem).wait()
      pltpu.sync_copy(rows, shared.at[idx.at[off + j, 1]], add=True)
      return carry

    lax.fori_loop(0, nch_my, body, 0)
    plsc.subcore_barrier()
    pltpu.sync_copy(shared.at[pl.ds(r0, RPT)],
                    out_hbm.at[c, pl.ds(r0, RPT)])

  return prop


def _make_prop(width, nbuf, a_chunks):
  ring = 2 * nbuf
  b_chunks = 2 * NCH - a_chunks

  if nbuf == 1:
    return _make_prop_simple(width, a_chunks, b_chunks)
  assert a_chunks % ring == 0 and b_chunks % ring == 0

  @functools.partial(
      pl.kernel,
      out_type=jax.ShapeDtypeStruct((2, NPAD, width), jnp.float32),
      mesh=_sc_mesh,
      compiler_params=_sc_params,
      scratch_types=(
          [pltpu.VMEM((ring, 2, CHUNK), jnp.int32)]
          + [pltpu.VMEM((CHUNK, width), jnp.float32)] * nbuf
          + [pltpu.VMEM_SHARED((NPAD, width), jnp.float32)]
          + [pltpu.SemaphoreType.DMA] * (ring + 2 * nbuf)
      ),
  )
  def prop(xp_hbm, edges_hbm, zeros_hbm, out_hbm, iring, *rest):
    rows = rest[:nbuf]
    shared = rest[nbuf]
    isem = rest[nbuf + 1:nbuf + 1 + ring]
    gsem = rest[nbuf + 1 + ring:nbuf + 1 + ring + nbuf]
    ssem = rest[nbuf + 1 + ring + nbuf:]
    c = lax.axis_index("c")
    s = lax.axis_index("s")
    r0 = pl.multiple_of(s * RPT, 8)
    base = jnp.where(c == 0, s * a_chunks, 16 * a_chunks + s * b_chunks)
    nch_my = jnp.where(c == 0, a_chunks, b_chunks)
    pltpu.sync_copy(zeros_hbm.at[pl.ds(r0, RPT)], shared.at[pl.ds(r0, RPT)])

    def fire_idx(k, r):
      pltpu.async_copy(edges_hbm.at[base + k], iring.at[r], isem[r])

    def wait_idx(k, r):
      pltpu.make_async_copy(edges_hbm.at[base + k], iring.at[r],
                            isem[r]).wait()

    def fire_gather(r, b):
      pltpu.async_copy(xp_hbm.at[iring.at[r, 0]], rows[b], gsem[b])

    plsc.subcore_barrier()
    for r in range(ring):
      fire_idx(r, r)
    for b in range(nbuf):
      wait_idx(b, b)
      fire_gather(b, b)

    def body(i, carry):
      K = i * ring
      for g in range(ring // nbuf):
        for b in range(nbuf):
          p = g * nbuf + b
          pltpu.make_async_copy(xp_hbm.at[iring.at[p, 0]], rows[b],
                                gsem[b]).wait()
          pltpu.async_copy(rows[b], shared.at[iring.at[p, 1]], ssem[b],
                           add=True)
        for b in range(nbuf):
          p = g * nbuf + b
          k = K + p
          pltpu.make_async_copy(rows[b], shared.at[iring.at[p, 1]],
                                ssem[b]).wait()

          @pl.when(k + ring < nch_my)
          def _():
            fire_idx(k + ring, p)

          @pl.when(k + nbuf < nch_my)
          def _():
            wait_idx(k + nbuf, (p + nbuf) % ring)
            fire_gather((p + nbuf) % ring, b)

      return carry

    lax.fori_loop(0, nch_my // ring, body, 0)
    plsc.subcore_barrier()
    pltpu.sync_copy(shared.at[pl.ds(r0, RPT)],
                    out_hbm.at[c, pl.ds(r0, RPT)])

  return prop


def _make_prop_pipe_staged(width, nbuf, chunk):
  nch = EPT // chunk
  assert nch * chunk == EPT and nch % nbuf == 0

  @functools.partial(
      pl.kernel,
      out_type=jax.ShapeDtypeStruct((2, NPAD, width), jnp.float32),
      mesh=_sc_mesh,
      compiler_params=_sc_params,
      scratch_types=(
          [pltpu.VMEM((nch, 2, chunk), jnp.int32)]
          + [pltpu.VMEM((chunk, width), jnp.float32)] * nbuf
          + [pltpu.VMEM_SHARED((NPAD, width), jnp.float32)]
          + [pltpu.SemaphoreType.DMA] * (2 * nbuf)
      ),
  )
  def prop(xp_hbm, edges_hbm, zeros_hbm, out_hbm, idx, *rest):
    rows = rest[:nbuf]
    shared = rest[nbuf]
    gsem = rest[nbuf + 1:nbuf + 1 + nbuf]
    ssem = rest[nbuf + 1 + nbuf:]
    c = lax.axis_index("c")
    s = lax.axis_index("s")
    wid = c * 16 + s
    r0 = pl.multiple_of(s * RPT, 8)
    pltpu.sync_copy(zeros_hbm.at[pl.ds(r0, RPT)], shared.at[pl.ds(r0, RPT)])
    pltpu.sync_copy(edges_hbm.at[wid], idx)
    plsc.subcore_barrier()

    for b in range(nbuf):
      pltpu.async_copy(xp_hbm.at[idx.at[b, 0]], rows[b], gsem[b])

    def body(j, carry):
      k0 = j * nbuf
      for b in range(nbuf):
        k = k0 + b
        pltpu.make_async_copy(xp_hbm.at[idx.at[k, 0]], rows[b],
                              gsem[b]).wait()
        pltpu.async_copy(rows[b], shared.at[idx.at[k, 1]], ssem[b],
                         add=True)
      for b in range(nbuf):
        k = k0 + b
        pltpu.make_async_copy(rows[b], shared.at[idx.at[k, 1]],
                              ssem[b]).wait()

        @pl.when(k + nbuf < nch)
        def _():
          pltpu.async_copy(xp_hbm.at[idx.at[k + nbuf, 0]], rows[b], gsem[b])

      return carry

    lax.fori_loop(0, nch // nbuf, body, 0)
    plsc.subcore_barrier()
    pltpu.sync_copy(shared.at[pl.ds(r0, RPT)],
                    out_hbm.at[c, pl.ds(r0, RPT)])

  return prop


def _make_prop128_colsplit(nbuf):
  ring = 2 * nbuf
  hw = D // 2
  tch = 2 * NCH
  assert tch % ring == 0

  @functools.partial(
      pl.kernel,
      out_type=jax.ShapeDtypeStruct((2, NPAD, hw), jnp.float32),
      mesh=_sc_mesh,
      compiler_params=_sc_params,
      scratch_types=(
          [pltpu.VMEM((ring, 2, CHUNK), jnp.int32)]
          + [pltpu.VMEM((CHUNK, hw), jnp.float32)] * nbuf
          + [pltpu.VMEM_SHARED((NPAD, hw), jnp.float32)] * 2
          + [pltpu.SemaphoreType.DMA] * (ring + 2 * nbuf)
      ),
  )
  def prop(xp_hbm, edges_hbm, zeros_hbm, out_hbm, iring, *rest):
    rows = rest[:nbuf]
    xp_sh = rest[nbuf]
    acc_sh = rest[nbuf + 1]
    isem = rest[nbuf + 2:nbuf + 2 + ring]
    gsem = rest[nbuf + 2 + ring:nbuf + 2 + ring + nbuf]
    ssem = rest[nbuf + 2 + ring + nbuf:]
    c = lax.axis_index("c")
    s = lax.axis_index("s")
    r0 = pl.multiple_of(s * RPT, 8)
    col = pl.multiple_of(c * hw, 8)
    base = s * tch
    pltpu.sync_copy(zeros_hbm.at[pl.ds(r0, RPT), pl.ds(0, hw)],
                    acc_sh.at[pl.ds(r0, RPT)])
    pltpu.sync_copy(xp_hbm.at[pl.ds(r0, RPT), pl.ds(col, hw)],
                    xp_sh.at[pl.ds(r0, RPT)])

    def fire_idx(k, r):
      pltpu.async_copy(edges_hbm.at[base + k], iring.at[r], isem[r])

    def wait_idx(k, r):
      pltpu.make_async_copy(edges_hbm.at[base + k], iring.at[r],
                            isem[r]).wait()

    def fire_gather(r, b):
      pltpu.async_copy(xp_sh.at[iring.at[r, 0]], rows[b], gsem[b])

    plsc.subcore_barrier()
    for r in range(ring):
      fire_idx(r, r)
    for b in range(nbuf):
      wait_idx(b, b)
      fire_gather(b, b)

    def body(i, carry):
      K = i * ring
      for g in range(ring // nbuf):
        for b in range(nbuf):
          p = g * nbuf + b
          pltpu.make_async_copy(xp_sh.at[iring.at[p, 0]], rows[b],
                                gsem[b]).wait()
          pltpu.async_copy(rows[b], acc_sh.at[iring.at[p, 1]], ssem[b],
                           add=True)
        for b in range(nbuf):
          p = g * nbuf + b
          k = K + p
          pltpu.make_async_copy(rows[b], acc_sh.at[iring.at[p, 1]],
                                ssem[b]).wait()

          @pl.when(k + ring < tch)
          def _():
            fire_idx(k + ring, p)

          @pl.when(k + nbuf < tch)
          def _():
            wait_idx(k + nbuf, (p + nbuf) % ring)
            fire_gather((p + nbuf) % ring, b)

      return carry

    lax.fori_loop(0, tch // ring, body, 0)
    plsc.subcore_barrier()
    pltpu.sync_copy(acc_sh.at[pl.ds(r0, RPT)],
                    out_hbm.at[c, pl.ds(r0, RPT)])

  return prop


def _make_prop16_spmem(nbuf):
  @functools.partial(
      pl.kernel,
      out_type=jax.ShapeDtypeStruct((2, NPAD, DL), jnp.float32),
      mesh=_sc_mesh,
      compiler_params=_sc_params,
      scratch_types=(
          [pltpu.VMEM((NCH, 2, CHUNK), jnp.int32)]
          + [pltpu.VMEM((CHUNK, DL), jnp.float32)] * nbuf
          + [pltpu.VMEM_SHARED((NPAD, DL), jnp.float32)] * 2
          + [pltpu.SemaphoreType.DMA] * (2 * nbuf)
      ),
  )
  def prop(xp_hbm, edges_hbm, zeros_hbm, out_hbm, idx, *rest):
    rows = rest[:nbuf]
    xp_sh = rest[nbuf]
    acc_sh = rest[nbuf + 1]
    gsem = rest[nbuf + 2:nbuf + 2 + nbuf]
    ssem = rest[nbuf + 2 + nbuf:]
    c = lax.axis_index("c")
    s = lax.axis_index("s")
    wid = c * 16 + s
    r0 = pl.multiple_of(s * RPT, 8)
    pltpu.sync_copy(zeros_hbm.at[pl.ds(r0, RPT)], acc_sh.at[pl.ds(r0, RPT)])
    pltpu.sync_copy(xp_hbm.at[pl.ds(r0, RPT)], xp_sh.at[pl.ds(r0, RPT)])
    pltpu.sync_copy(edges_hbm.at[pl.ds(wid * NCH, NCH)], idx)
    plsc.subcore_barrier()

    for b in range(nbuf):
      pltpu.async_copy(xp_sh.at[idx.at[b, 0]], rows[b], gsem[b])

    def body(j, carry):
      k0 = j * nbuf
      for b in range(nbuf):
        k = k0 + b
        pltpu.make_async_copy(xp_sh.at[idx.at[k, 0]], rows[b],
                              gsem[b]).wait()
        pltpu.async_copy(rows[b], acc_sh.at[idx.at[k, 1]], ssem[b],
                         add=True)
      for b in range(nbuf):
        k = k0 + b
        pltpu.make_async_copy(rows[b], acc_sh.at[idx.at[k, 1]],
                              ssem[b]).wait()

        @pl.when(k + nbuf < NCH)
        def _():
          pltpu.async_copy(xp_sh.at[idx.at[k + nbuf, 0]], rows[b], gsem[b])

      return carry

    lax.fori_loop(0, NCH // nbuf, body, 0)
    plsc.subcore_barrier()
    pltpu.sync_copy(acc_sh.at[pl.ds(r0, RPT)],
                    out_hbm.at[c, pl.ds(r0, RPT)])

  return prop


_prop128 = _make_prop128_colsplit(4)
_prop16 = _make_prop16_spmem(4)


@functools.partial(
    pl.kernel,
    out_type=jax.ShapeDtypeStruct((2, NPAD, DL), jnp.float32),
    mesh=_sc_mesh,
    compiler_params=_sc_params,
    scratch_types=(
        [pltpu.VMEM((NCH, CHUNK), jnp.int32),
         pltpu.VMEM((CHUNK, DL), jnp.float32),
         pltpu.VMEM_SHARED((NPAD, DL), jnp.float32)]
        + [pltpu.SemaphoreType.DMA] * 8
    ),
)
def _deg_kernel(dst_hbm, zeros_hbm, ones_hbm, out_hbm,
                dstbuf, onesv, shared, *sems):
  c = lax.axis_index("c")
  s = lax.axis_index("s")
  wid = c * 16 + s
  r0 = pl.multiple_of(s * RPT, 8)
  pltpu.sync_copy(zeros_hbm.at[pl.ds(r0, RPT)], shared.at[pl.ds(r0, RPT)])
  pltpu.sync_copy(ones_hbm, onesv)
  pltpu.sync_copy(dst_hbm.at[wid], dstbuf)
  plsc.subcore_barrier()

  def body(j, carry):
    for b in range(8):
      pltpu.async_copy(onesv, shared.at[dstbuf.at[j * 8 + b]], sems[b],
                       add=True)
    for b in range(8):
      pltpu.make_async_copy(onesv, shared.at[dstbuf.at[j * 8 + b]],
                            sems[b]).wait()
    return carry

  lax.fori_loop(0, NCH // 8, body, 0)
  plsc.subcore_barrier()
  pltpu.sync_copy(shared.at[pl.ds(r0, RPT)],
                  out_hbm.at[c, pl.ds(r0, RPT)])


@functools.partial(
    pl.kernel,
    out_type=jax.ShapeDtypeStruct((NIDPAD, 2 * DL), jnp.float32),
    mesh=_sc_mesh,
    compiler_params=_sc_params,
    scratch_types=[
        pltpu.VMEM((IDS_PT,), jnp.int32),
        pltpu.VMEM((IDS_PT, 2 * DL), jnp.float32),
        pltpu.SemaphoreType.DMA,
    ],
)
def _gather_kernel(comb_hbm, ids_hbm, out_hbm, idbuf, rowsbuf, sem):
  c = lax.axis_index("c")
  s = lax.axis_index("s")
  wid = c * 16 + s
  pltpu.sync_copy(ids_hbm.at[wid], idbuf)
  pltpu.async_copy(comb_hbm.at[idbuf], rowsbuf, sem).wait()
  pltpu.sync_copy(rowsbuf, out_hbm.at[pl.ds(pl.multiple_of(wid * IDS_PT, 8),
                                            IDS_PT)])


BR = 640


def _mm1_body(x_ref, w_ref, degp_ref, xp_ref, dinv_ref):
  deg = degp_ref[0] + degp_ref[1] + 1.0
  dinv = lax.rsqrt(jnp.maximum(deg, 1.0))
  h = jnp.dot(x_ref[...], w_ref[...], preferred_element_type=jnp.float32)
  xp_ref[...] = h * dinv[:, :1]
  dinv_ref[...] = dinv


_mm1 = pl.pallas_call(
    _mm1_body,
    grid=(NPAD // BR,),
    in_specs=[
        pl.BlockSpec((BR, D), lambda i: (i, 0)),
        pl.BlockSpec((D, D), lambda i: (0, 0)),
        pl.BlockSpec((2, BR, DL), lambda i: (0, i, 0)),
    ],
    out_specs=[
        pl.BlockSpec((BR, D), lambda i: (i, 0)),
        pl.BlockSpec((BR, DL), lambda i: (i, 0)),
    ],
    out_shape=[
        jax.ShapeDtypeStruct((NPAD, D), jnp.float32),
        jax.ShapeDtypeStruct((NPAD, DL), jnp.float32),
    ],
)


def _mid_body(s1p_ref, xp1_ref, dinv_ref, b1_ref, w2_ref, xp2_ref):
  dinv = dinv_ref[...][:, :1]
  s1 = jnp.concatenate([s1p_ref[0], s1p_ref[1]], axis=1)
  out1 = (s1 + xp1_ref[...]) * dinv + b1_ref[...]
  h1 = jnp.maximum(out1, 0.0)
  xp2_ref[...] = jnp.dot(h1, w2_ref[...],
                         preferred_element_type=jnp.float32) * dinv


_mid = pl.pallas_call(
    _mid_body,
    grid=(NPAD // BR,),
    in_specs=[
        pl.BlockSpec((2, BR, D // 2), lambda i: (0, i, 0)),
        pl.BlockSpec((BR, D), lambda i: (i, 0)),
        pl.BlockSpec((BR, DL), lambda i: (i, 0)),
        pl.BlockSpec((1, D), lambda i: (0, 0)),
        pl.BlockSpec((D, DL), lambda i: (0, 0)),
    ],
    out_specs=pl.BlockSpec((BR, DL), lambda i: (i, 0)),
    out_shape=jax.ShapeDtypeStruct((NPAD, DL), jnp.float32),
)


def _pack_body(s2p_ref, xp2_ref, dinv_ref, b2_ref, lab_ref, comb_ref):
  dinv = dinv_ref[...][:, :1]
  out2 = (s2p_ref[0] + s2p_ref[1] + xp2_ref[...]) * dinv + b2_ref[...]
  labf = lax.bitcast_convert_type(lab_ref[...], jnp.float32)
  comb_ref[...] = jnp.concatenate([out2, labf], axis=1)


_pack = pl.pallas_call(
    _pack_body,
    grid=(NPAD // BR,),
    in_specs=[
        pl.BlockSpec((2, BR, DL), lambda i: (0, i, 0)),
        pl.BlockSpec((BR, DL), lambda i: (i, 0)),
        pl.BlockSpec((BR, DL), lambda i: (i, 0)),
        pl.BlockSpec((1, DL), lambda i: (0, 0)),
        pl.BlockSpec((BR, DL), lambda i: (i, 0)),
    ],
    out_specs=pl.BlockSpec((BR, 2 * DL), lambda i: (i, 0)),
    out_shape=jax.ShapeDtypeStruct((NPAD, 2 * DL), jnp.float32),
)


def _head_body(comb_ref, loss_ref):
  z = comb_ref[...]
  logits = z[:, :DL]
  lab = lax.bitcast_convert_type(z[:, DL:], jnp.int32)
  col = lax.broadcasted_iota(jnp.int32, (NIDPAD, DL), 1)
  picked = jnp.sum(jnp.where(col == lab, logits, 0.0), axis=1, keepdims=True)
  m = jnp.max(logits, axis=1, keepdims=True)
  lse = jnp.log(jnp.sum(jnp.exp(logits - m), axis=1, keepdims=True)) + m
  rowi = lax.broadcasted_iota(jnp.int32, (NIDPAD, 1), 0)
  mask = jnp.where(rowi < NID, 1.0, 0.0)
  loss_ref[...] = jnp.reshape(jnp.sum((lse - picked) * mask) / NID, (1, 1))


_head = pl.pallas_call(
    _head_body,
    grid=(1,),
    in_specs=[pl.BlockSpec((NIDPAD, 2 * DL), lambda i: (0, 0))],
    out_specs=pl.BlockSpec((1, 1), lambda i: (0, 0)),
    out_shape=jax.ShapeDtypeStruct((1, 1), jnp.float32),
)


def kernel(x, edge_index, node_ids, label_inds, W1, b1, W2, b2):
  x_pad = jnp.concatenate([x, jnp.zeros((NPAD - N, D), jnp.float32)], axis=0)
  pad_src = jnp.zeros((EPAD - E,), jnp.int32)
  pad_dst = (N + (jnp.arange(EPAD - E, dtype=jnp.int32) % (NPAD - N)))
  src_flat = jnp.concatenate([edge_index[0], pad_src])
  dst_flat = jnp.concatenate([edge_index[1], pad_dst])
  edges_il = jnp.stack([src_flat.reshape(-1, CHUNK),
                        dst_flat.reshape(-1, CHUNK)], axis=1)
  dst = dst_flat.reshape(NW, NCH, CHUNK)
  ids = jnp.concatenate(
      [node_ids.astype(jnp.int32),
       jnp.zeros((NIDPAD - NID,), jnp.int32)]).reshape(NW, IDS_PT)
  lab_pad = jnp.concatenate(
      [label_inds.astype(jnp.int32), jnp.zeros((NPAD - N,), jnp.int32)])
  lab16 = jnp.broadcast_to(lab_pad[:, None], (NPAD, DL))
  zeros128 = jnp.zeros((NPAD, D), jnp.float32)
  zeros16 = jnp.zeros((NPAD, DL), jnp.float32)
  ones16 = jnp.ones((CHUNK, DL), jnp.float32)

  degp = _deg_kernel(dst, zeros16, ones16)
  xp1, dinv16 = _mm1(x_pad, W1, degp)
  s1p = _prop128(xp1, edges_il, zeros128)
  xp2 = _mid(s1p, xp1, dinv16, b1.reshape(1, D), W2)
  s2p = _prop16(xp2, edges_il, zeros16)
  comb = _pack(s2p, xp2, dinv16, b2.reshape(1, DL), lab16)
  g = _gather_kernel(comb, ids)
  lossm = _head(g)
  return (lossm[0, 0], g[:NID, :DL])


# speedup vs baseline: 1.1042x; 1.1042x over previous
import functools

import jax
import jax.numpy as jnp
from jax import lax
from jax.experimental import pallas as pl
from jax.experimental.pallas import tpu as pltpu
from jax.experimental.pallas import tpu_sc as plsc

N = 10000
E = 320000
D = 128
DL = 16
NID = 2000

NPAD = 10240
NW = 32
CHUNK = 128
NCH = 80
EPT = NCH * CHUNK
EPAD = NW * EPT
RPT = NPAD // 16
NIDPAD = 2048
IDS_PT = NIDPAD // NW

_sc_mesh = plsc.VectorSubcoreMesh(core_axis_name="c", subcore_axis_name="s")
_sc_params = pltpu.CompilerParams(use_tc_tiling_on_sc=False)


def _make_prop_simple(width, a_chunks, b_chunks):
  bmax = max(a_chunks, b_chunks)

  @functools.partial(
      pl.kernel,
      out_type=jax.ShapeDtypeStruct((2, NPAD, width), jnp.float32),
      mesh=_sc_mesh,
      compiler_params=_sc_params,
      scratch_types=[
          pltpu.VMEM((bmax, 2, CHUNK), jnp.int32),
          pltpu.VMEM((CHUNK, width), jnp.float32),
          pltpu.VMEM_SHARED((NPAD, width), jnp.float32),
          pltpu.SemaphoreType.DMA,
      ],
  )
  def prop(xp_hbm, edges_hbm, zeros_hbm, out_hbm, idx, rows, shared, sem):
    c = lax.axis_index("c")
    s = lax.axis_index("s")
    r0 = pl.multiple_of(s * RPT, 8)
    base = jnp.where(c == 0, s * a_chunks, 16 * a_chunks + s * b_chunks)
    nch_my = jnp.where(c == 0, a_chunks, b_chunks)
    base = jnp.minimum(base, 2 * 16 * NCH - bmax)
    pltpu.sync_copy(zeros_hbm.at[pl.ds(r0, RPT)], shared.at[pl.ds(r0, RPT)])
    pltpu.sync_copy(edges_hbm.at[pl.ds(base, bmax)], idx)
    off = jnp.where(c == 0, s * a_chunks - base, 16 * a_chunks + s * b_chunks - base)
    plsc.subcore_barrier()

    def body(j, carry):
      pltpu.async_copy(xp_hbm.at[idx.at[off + j, 0]], rows, sem).wait()
      pltpu.sync_copy(rows, shared.at[idx.at[off + j, 1]], add=True)
      return carry

    lax.fori_loop(0, nch_my, body, 0)
    plsc.subcore_barrier()
    pltpu.sync_copy(shared.at[pl.ds(r0, RPT)],
                    out_hbm.at[c, pl.ds(r0, RPT)])

  return prop


def _make_prop(width, nbuf, a_chunks):
  ring = 2 * nbuf
  b_chunks = 2 * NCH - a_chunks

  if nbuf == 1:
    return _make_prop_simple(width, a_chunks, b_chunks)
  assert a_chunks % ring == 0 and b_chunks % ring == 0

  @functools.partial(
      pl.kernel,
      out_type=jax.ShapeDtypeStruct((2, NPAD, width), jnp.float32),
      mesh=_sc_mesh,
      compiler_params=_sc_params,
      scratch_types=(
          [pltpu.VMEM((ring, 2, CHUNK), jnp.int32)]
          + [pltpu.VMEM((CHUNK, width), jnp.float32)] * nbuf
          + [pltpu.VMEM_SHARED((NPAD, width), jnp.float32)]
          + [pltpu.SemaphoreType.DMA] * (ring + 2 * nbuf)
      ),
  )
  def prop(xp_hbm, edges_hbm, zeros_hbm, out_hbm, iring, *rest):
    rows = rest[:nbuf]
    shared = rest[nbuf]
    isem = rest[nbuf + 1:nbuf + 1 + ring]
    gsem = rest[nbuf + 1 + ring:nbuf + 1 + ring + nbuf]
    ssem = rest[nbuf + 1 + ring + nbuf:]
    c = lax.axis_index("c")
    s = lax.axis_index("s")
    r0 = pl.multiple_of(s * RPT, 8)
    base = jnp.where(c == 0, s * a_chunks, 16 * a_chunks + s * b_chunks)
    nch_my = jnp.where(c == 0, a_chunks, b_chunks)
    pltpu.sync_copy(zeros_hbm.at[pl.ds(r0, RPT)], shared.at[pl.ds(r0, RPT)])

    def fire_idx(k, r):
      pltpu.async_copy(edges_hbm.at[base + k], iring.at[r], isem[r])

    def wait_idx(k, r):
      pltpu.make_async_copy(edges_hbm.at[base + k], iring.at[r],
                            isem[r]).wait()

    def fire_gather(r, b):
      pltpu.async_copy(xp_hbm.at[iring.at[r, 0]], rows[b], gsem[b])

    plsc.subcore_barrier()
    for r in range(ring):
      fire_idx(r, r)
    for b in range(nbuf):
      wait_idx(b, b)
      fire_gather(b, b)

    def body(i, carry):
      K = i * ring
      for g in range(ring // nbuf):
        for b in range(nbuf):
          p = g * nbuf + b
          pltpu.make_async_copy(xp_hbm.at[iring.at[p, 0]], rows[b],
                                gsem[b]).wait()
          pltpu.async_copy(rows[b], shared.at[iring.at[p, 1]], ssem[b],
                           add=True)
        for b in range(nbuf):
          p = g * nbuf + b
          k = K + p
          pltpu.make_async_copy(rows[b], shared.at[iring.at[p, 1]],
                                ssem[b]).wait()

          @pl.when(k + ring < nch_my)
          def _():
            fire_idx(k + ring, p)

          @pl.when(k + nbuf < nch_my)
          def _():
            wait_idx(k + nbuf, (p + nbuf) % ring)
            fire_gather((p + nbuf) % ring, b)

      return carry

    lax.fori_loop(0, nch_my // ring, body, 0)
    plsc.subcore_barrier()
    pltpu.sync_copy(shared.at[pl.ds(r0, RPT)],
                    out_hbm.at[c, pl.ds(r0, RPT)])

  return prop


def _make_prop_pipe_staged(width, nbuf, chunk):
  nch = EPT // chunk
  assert nch * chunk == EPT and nch % nbuf == 0

  @functools.partial(
      pl.kernel,
      out_type=jax.ShapeDtypeStruct((2, NPAD, width), jnp.float32),
      mesh=_sc_mesh,
      compiler_params=_sc_params,
      scratch_types=(
          [pltpu.VMEM((nch, 2, chunk), jnp.int32)]
          + [pltpu.VMEM((chunk, width), jnp.float32)] * nbuf
          + [pltpu.VMEM_SHARED((NPAD, width), jnp.float32)]
          + [pltpu.SemaphoreType.DMA] * (2 * nbuf)
      ),
  )
  def prop(xp_hbm, edges_hbm, zeros_hbm, out_hbm, idx, *rest):
    rows = rest[:nbuf]
    shared = rest[nbuf]
    gsem = rest[nbuf + 1:nbuf + 1 + nbuf]
    ssem = rest[nbuf + 1 + nbuf:]
    c = lax.axis_index("c")
    s = lax.axis_index("s")
    wid = c * 16 + s
    r0 = pl.multiple_of(s * RPT, 8)
    pltpu.sync_copy(zeros_hbm.at[pl.ds(r0, RPT)], shared.at[pl.ds(r0, RPT)])
    pltpu.sync_copy(edges_hbm.at[wid], idx)
    plsc.subcore_barrier()

    for b in range(nbuf):
      pltpu.async_copy(xp_hbm.at[idx.at[b, 0]], rows[b], gsem[b])

    def body(j, carry):
      k0 = j * nbuf
      for b in range(nbuf):
        k = k0 + b
        pltpu.make_async_copy(xp_hbm.at[idx.at[k, 0]], rows[b],
                              gsem[b]).wait()
        pltpu.async_copy(rows[b], shared.at[idx.at[k, 1]], ssem[b],
                         add=True)
      for b in range(nbuf):
        k = k0 + b
        pltpu.make_async_copy(rows[b], shared.at[idx.at[k, 1]],
                              ssem[b]).wait()

        @pl.when(k + nbuf < nch)
        def _():
          pltpu.async_copy(xp_hbm.at[idx.at[k + nbuf, 0]], rows[b], gsem[b])

      return carry

    lax.fori_loop(0, nch // nbuf, body, 0)
    plsc.subcore_barrier()
    pltpu.sync_copy(shared.at[pl.ds(r0, RPT)],
                    out_hbm.at[c, pl.ds(r0, RPT)])

  return prop


def _make_prop128_colsplit(nbuf):
  ring = 2 * nbuf
  hw = D // 2
  tch = 2 * NCH
  assert tch % ring == 0

  @functools.partial(
      pl.kernel,
      out_type=jax.ShapeDtypeStruct((2, NPAD, hw), jnp.float32),
      mesh=_sc_mesh,
      compiler_params=_sc_params,
      scratch_types=(
          [pltpu.VMEM((ring, 2, CHUNK), jnp.int32)]
          + [pltpu.VMEM((CHUNK, hw), jnp.float32)] * nbuf
          + [pltpu.VMEM_SHARED((NPAD, hw), jnp.float32)] * 2
          + [pltpu.SemaphoreType.DMA] * (ring + 2 * nbuf)
      ),
  )
  def prop(xp_hbm, edges_hbm, zeros_hbm, out_hbm, iring, *rest):
    rows = rest[:nbuf]
    xp_sh = rest[nbuf]
    acc_sh = rest[nbuf + 1]
    isem = rest[nbuf + 2:nbuf + 2 + ring]
    gsem = rest[nbuf + 2 + ring:nbuf + 2 + ring + nbuf]
    ssem = rest[nbuf + 2 + ring + nbuf:]
    c = lax.axis_index("c")
    s = lax.axis_index("s")
    r0 = pl.multiple_of(s * RPT, 8)
    col = pl.multiple_of(c * hw, 8)
    base = s * tch
    pltpu.sync_copy(zeros_hbm.at[pl.ds(r0, RPT), pl.ds(0, hw)],
                    acc_sh.at[pl.ds(r0, RPT)])
    pltpu.sync_copy(xp_hbm.at[pl.ds(r0, RPT), pl.ds(col, hw)],
                    xp_sh.at[pl.ds(r0, RPT)])

    def fire_idx(k, r):
      pltpu.async_copy(edges_hbm.at[base + k], iring.at[r], isem[r])

    def wait_idx(k, r):
      pltpu.make_async_copy(edges_hbm.at[base + k], iring.at[r],
                            isem[r]).wait()

    def fire_gather(r, b):
      pltpu.async_copy(xp_sh.at[iring.at[r, 0]], rows[b], gsem[b])

    plsc.subcore_barrier()
    for r in range(ring):
      fire_idx(r, r)
    for b in range(nbuf):
      wait_idx(b, b)
      fire_gather(b, b)

    def body(i, carry):
      K = i * ring
      for g in range(ring // nbuf):
        for b in range(nbuf):
          p = g * nbuf + b
          pltpu.make_async_copy(xp_sh.at[iring.at[p, 0]], rows[b],
                                gsem[b]).wait()
          pltpu.async_copy(rows[b], acc_sh.at[iring.at[p, 1]], ssem[b],
                           add=True)
        for b in range(nbuf):
          p = g * nbuf + b
          k = K + p
          pltpu.make_async_copy(rows[b], acc_sh.at[iring.at[p, 1]],
                                ssem[b]).wait()

          @pl.when(k + ring < tch)
          def _():
            fire_idx(k + ring, p)

          @pl.when(k + nbuf < tch)
          def _():
            wait_idx(k + nbuf, (p + nbuf) % ring)
            fire_gather((p + nbuf) % ring, b)

      return carry

    lax.fori_loop(0, tch // ring, body, 0)
    plsc.subcore_barrier()
    pltpu.sync_copy(acc_sh.at[pl.ds(r0, RPT)],
                    out_hbm.at[c, pl.ds(r0, RPT)])

  return prop


def _make_prop16_spmem(nbuf):
  @functools.partial(
      pl.kernel,
      out_type=jax.ShapeDtypeStruct((2, NPAD, DL), jnp.float32),
      mesh=_sc_mesh,
      compiler_params=_sc_params,
      scratch_types=(
          [pltpu.VMEM((NCH, 2, CHUNK), jnp.int32)]
          + [pltpu.VMEM((CHUNK, DL), jnp.float32)] * nbuf
          + [pltpu.VMEM_SHARED((NPAD, DL), jnp.float32)] * 2
          + [pltpu.SemaphoreType.DMA] * (2 * nbuf)
      ),
  )
  def prop(xp_hbm, edges_hbm, zeros_hbm, out_hbm, idx, *rest):
    rows = rest[:nbuf]
    xp_sh = rest[nbuf]
    acc_sh = rest[nbuf + 1]
    gsem = rest[nbuf + 2:nbuf + 2 + nbuf]
    ssem = rest[nbuf + 2 + nbuf:]
    c = lax.axis_index("c")
    s = lax.axis_index("s")
    wid = c * 16 + s
    r0 = pl.multiple_of(s * RPT, 8)
    pltpu.sync_copy(zeros_hbm.at[pl.ds(r0, RPT)], acc_sh.at[pl.ds(r0, RPT)])
    pltpu.sync_copy(xp_hbm.at[pl.ds(r0, RPT)], xp_sh.at[pl.ds(r0, RPT)])
    pltpu.sync_copy(edges_hbm.at[pl.ds(wid * NCH, NCH)], idx)
    plsc.subcore_barrier()

    for b in range(nbuf):
      pltpu.async_copy(xp_sh.at[idx.at[b, 0]], rows[b], gsem[b])

    def body(j, carry):
      k0 = j * nbuf
      for b in range(nbuf):
        k = k0 + b
        pltpu.make_async_copy(xp_sh.at[idx.at[k, 0]], rows[b],
                              gsem[b]).wait()
        pltpu.async_copy(rows[b], acc_sh.at[idx.at[k, 1]], ssem[b],
                         add=True)
      for b in range(nbuf):
        k = k0 + b
        pltpu.make_async_copy(rows[b], acc_sh.at[idx.at[k, 1]],
                              ssem[b]).wait()

        @pl.when(k + nbuf < NCH)
        def _():
          pltpu.async_copy(xp_sh.at[idx.at[k + nbuf, 0]], rows[b], gsem[b])

      return carry

    lax.fori_loop(0, NCH // nbuf, body, 0)
    plsc.subcore_barrier()
    pltpu.sync_copy(acc_sh.at[pl.ds(r0, RPT)],
                    out_hbm.at[c, pl.ds(r0, RPT)])

  return prop


_prop128 = _make_prop128_colsplit(2)
_prop16 = _make_prop16_spmem(2)


@functools.partial(
    pl.kernel,
    out_type=jax.ShapeDtypeStruct((2, NPAD, DL), jnp.float32),
    mesh=_sc_mesh,
    compiler_params=_sc_params,
    scratch_types=(
        [pltpu.VMEM((NCH, CHUNK), jnp.int32),
         pltpu.VMEM((CHUNK, DL), jnp.float32),
         pltpu.VMEM_SHARED((NPAD, DL), jnp.float32)]
        + [pltpu.SemaphoreType.DMA] * 8
    ),
)
def _deg_kernel(dst_hbm, zeros_hbm, ones_hbm, out_hbm,
                dstbuf, onesv, shared, *sems):
  c = lax.axis_index("c")
  s = lax.axis_index("s")
  wid = c * 16 + s
  r0 = pl.multiple_of(s * RPT, 8)
  pltpu.sync_copy(zeros_hbm.at[pl.ds(r0, RPT)], shared.at[pl.ds(r0, RPT)])
  pltpu.sync_copy(ones_hbm, onesv)
  pltpu.sync_copy(dst_hbm.at[wid], dstbuf)
  plsc.subcore_barrier()

  def body(j, carry):
    for b in range(8):
      pltpu.async_copy(onesv, shared.at[dstbuf.at[j * 8 + b]], sems[b],
                       add=True)
    for b in range(8):
      pltpu.make_async_copy(onesv, shared.at[dstbuf.at[j * 8 + b]],
                            sems[b]).wait()
    return carry

  lax.fori_loop(0, NCH // 8, body, 0)
  plsc.subcore_barrier()
  pltpu.sync_copy(shared.at[pl.ds(r0, RPT)],
                  out_hbm.at[c, pl.ds(r0, RPT)])


@functools.partial(
    pl.kernel,
    out_type=jax.ShapeDtypeStruct((NIDPAD, 2 * DL), jnp.float32),
    mesh=_sc_mesh,
    compiler_params=_sc_params,
    scratch_types=[
        pltpu.VMEM((IDS_PT,), jnp.int32),
        pltpu.VMEM((IDS_PT, 2 * DL), jnp.float32),
        pltpu.SemaphoreType.DMA,
    ],
)
def _gather_kernel(comb_hbm, ids_hbm, out_hbm, idbuf, rowsbuf, sem):
  c = lax.axis_index("c")
  s = lax.axis_index("s")
  wid = c * 16 + s
  pltpu.sync_copy(ids_hbm.at[wid], idbuf)
  pltpu.async_copy(comb_hbm.at[idbuf], rowsbuf, sem).wait()
  pltpu.sync_copy(rowsbuf, out_hbm.at[pl.ds(pl.multiple_of(wid * IDS_PT, 8),
                                            IDS_PT)])


BR = 640


def _mm1_body(x_ref, w_ref, degp_ref, xp_ref, dinv_ref):
  deg = degp_ref[0] + degp_ref[1] + 1.0
  dinv = lax.rsqrt(jnp.maximum(deg, 1.0))
  h = jnp.dot(x_ref[...], w_ref[...], preferred_element_type=jnp.float32)
  xp_ref[...] = h * dinv[:, :1]
  dinv_ref[...] = dinv


_mm1 = pl.pallas_call(
    _mm1_body,
    grid=(NPAD // BR,),
    in_specs=[
        pl.BlockSpec((BR, D), lambda i: (i, 0)),
        pl.BlockSpec((D, D), lambda i: (0, 0)),
        pl.BlockSpec((2, BR, DL), lambda i: (0, i, 0)),
    ],
    out_specs=[
        pl.BlockSpec((BR, D), lambda i: (i, 0)),
        pl.BlockSpec((BR, DL), lambda i: (i, 0)),
    ],
    out_shape=[
        jax.ShapeDtypeStruct((NPAD, D), jnp.float32),
        jax.ShapeDtypeStruct((NPAD, DL), jnp.float32),
    ],
)


def _mid_body(s1p_ref, xp1_ref, dinv_ref, b1_ref, w2_ref, xp2_ref):
  dinv = dinv_ref[...][:, :1]
  s1 = jnp.concatenate([s1p_ref[0], s1p_ref[1]], axis=1)
  out1 = (s1 + xp1_ref[...]) * dinv + b1_ref[...]
  h1 = jnp.maximum(out1, 0.0)
  xp2_ref[...] = jnp.dot(h1, w2_ref[...],
                         preferred_element_type=jnp.float32) * dinv


_mid = pl.pallas_call(
    _mid_body,
    grid=(NPAD // BR,),
    in_specs=[
        pl.BlockSpec((2, BR, D // 2), lambda i: (0, i, 0)),
        pl.BlockSpec((BR, D), lambda i: (i, 0)),
        pl.BlockSpec((BR, DL), lambda i: (i, 0)),
        pl.BlockSpec((1, D), lambda i: (0, 0)),
        pl.BlockSpec((D, DL), lambda i: (0, 0)),
    ],
    out_specs=pl.BlockSpec((BR, DL), lambda i: (i, 0)),
    out_shape=jax.ShapeDtypeStruct((NPAD, DL), jnp.float32),
)


def _pack_body(s2p_ref, xp2_ref, dinv_ref, b2_ref, lab_ref, comb_ref):
  dinv = dinv_ref[...][:, :1]
  out2 = (s2p_ref[0] + s2p_ref[1] + xp2_ref[...]) * dinv + b2_ref[...]
  labf = lax.bitcast_convert_type(lab_ref[...], jnp.float32)
  comb_ref[...] = jnp.concatenate([out2, labf], axis=1)


_pack = pl.pallas_call(
    _pack_body,
    grid=(NPAD // BR,),
    in_specs=[
        pl.BlockSpec((2, BR, DL), lambda i: (0, i, 0)),
        pl.BlockSpec((BR, DL), lambda i: (i, 0)),
        pl.BlockSpec((BR, DL), lambda i: (i, 0)),
        pl.BlockSpec((1, DL), lambda i: (0, 0)),
        pl.BlockSpec((BR, DL), lambda i: (i, 0)),
    ],
    out_specs=pl.BlockSpec((BR, 2 * DL), lambda i: (i, 0)),
    out_shape=jax.ShapeDtypeStruct((NPAD, 2 * DL), jnp.float32),
)


def _head_body(comb_ref, loss_ref):
  z = comb_ref[...]
  logits = z[:, :DL]
  lab = lax.bitcast_convert_type(z[:, DL:], jnp.int32)
  col = lax.broadcasted_iota(jnp.int32, (NIDPAD, DL), 1)
  picked = jnp.sum(jnp.where(col == lab, logits, 0.0), axis=1, keepdims=True)
  m = jnp.max(logits, axis=1, keepdims=True)
  lse = jnp.log(jnp.sum(jnp.exp(logits - m), axis=1, keepdims=True)) + m
  rowi = lax.broadcasted_iota(jnp.int32, (NIDPAD, 1), 0)
  mask = jnp.where(rowi < NID, 1.0, 0.0)
  loss_ref[...] = jnp.reshape(jnp.sum((lse - picked) * mask) / NID, (1, 1))


_head = pl.pallas_call(
    _head_body,
    grid=(1,),
    in_specs=[pl.BlockSpec((NIDPAD, 2 * DL), lambda i: (0, 0))],
    out_specs=pl.BlockSpec((1, 1), lambda i: (0, 0)),
    out_shape=jax.ShapeDtypeStruct((1, 1), jnp.float32),
)


def kernel(x, edge_index, node_ids, label_inds, W1, b1, W2, b2):
  x_pad = jnp.concatenate([x, jnp.zeros((NPAD - N, D), jnp.float32)], axis=0)
  pad_src = jnp.zeros((EPAD - E,), jnp.int32)
  pad_dst = (N + (jnp.arange(EPAD - E, dtype=jnp.int32) % (NPAD - N)))
  src_flat = jnp.concatenate([edge_index[0], pad_src])
  dst_flat = jnp.concatenate([edge_index[1], pad_dst])
  edges_il = jnp.stack([src_flat.reshape(-1, CHUNK),
                        dst_flat.reshape(-1, CHUNK)], axis=1)
  dst = dst_flat.reshape(NW, NCH, CHUNK)
  ids = jnp.concatenate(
      [node_ids.astype(jnp.int32),
       jnp.zeros((NIDPAD - NID,), jnp.int32)]).reshape(NW, IDS_PT)
  lab_pad = jnp.concatenate(
      [label_inds.astype(jnp.int32), jnp.zeros((NPAD - N,), jnp.int32)])
  lab16 = jnp.broadcast_to(lab_pad[:, None], (NPAD, DL))
  zeros128 = jnp.zeros((NPAD, D), jnp.float32)
  zeros16 = jnp.zeros((NPAD, DL), jnp.float32)
  ones16 = jnp.ones((CHUNK, DL), jnp.float32)

  degp = _deg_kernel(dst, zeros16, ones16)
  xp1, dinv16 = _mm1(x_pad, W1, degp)
  s1p = _prop128(xp1, edges_il, zeros128)
  xp2 = _mid(s1p, xp1, dinv16, b1.reshape(1, D), W2)
  s2p = _prop16(xp2, edges_il, zeros16)
  comb = _pack(s2p, xp2, dinv16, b2.reshape(1, DL), lab16)
  g = _gather_kernel(comb, ids)
  lossm = _head(g)
  return (lossm[0, 0], g[:NID, :DL])


# speedup vs baseline: 1.1177x; 1.0122x over previous
import functools

import jax
import jax.numpy as jnp
from jax import lax
from jax.experimental import pallas as pl
from jax.experimental.pallas import tpu as pltpu
from jax.experimental.pallas import tpu_sc as plsc

N = 10000
E = 320000
D = 128
DL = 16
NID = 2000

NPAD = 10240
NW = 32
CHUNK = 128
NCH = 80
EPT = NCH * CHUNK
EPAD = NW * EPT
RPT = NPAD // 16
NIDPAD = 2048
IDS_PT = NIDPAD // NW

_sc_mesh = plsc.VectorSubcoreMesh(core_axis_name="c", subcore_axis_name="s")
_sc_params = pltpu.CompilerParams(use_tc_tiling_on_sc=False)


def _make_prop_simple(width, a_chunks, b_chunks):
  bmax = max(a_chunks, b_chunks)

  @functools.partial(
      pl.kernel,
      out_type=jax.ShapeDtypeStruct((2, NPAD, width), jnp.float32),
      mesh=_sc_mesh,
      compiler_params=_sc_params,
      scratch_types=[
          pltpu.VMEM((bmax, 2, CHUNK), jnp.int32),
          pltpu.VMEM((CHUNK, width), jnp.float32),
          pltpu.VMEM_SHARED((NPAD, width), jnp.float32),
          pltpu.SemaphoreType.DMA,
      ],
  )
  def prop(xp_hbm, edges_hbm, zeros_hbm, out_hbm, idx, rows, shared, sem):
    c = lax.axis_index("c")
    s = lax.axis_index("s")
    r0 = pl.multiple_of(s * RPT, 8)
    base = jnp.where(c == 0, s * a_chunks, 16 * a_chunks + s * b_chunks)
    nch_my = jnp.where(c == 0, a_chunks, b_chunks)
    base = jnp.minimum(base, 2 * 16 * NCH - bmax)
    pltpu.sync_copy(zeros_hbm.at[pl.ds(r0, RPT)], shared.at[pl.ds(r0, RPT)])
    pltpu.sync_copy(edges_hbm.at[pl.ds(base, bmax)], idx)
    off = jnp.where(c == 0, s * a_chunks - base, 16 * a_chunks + s * b_chunks - base)
    plsc.subcore_barrier()

    def body(j, carry):
      pltpu.async_copy(xp_hbm.at[idx.at[off + j, 0]], rows, sem).wait()
      pltpu.sync_copy(rows, shared.at[idx.at[off + j, 1]], add=True)
      return carry

    lax.fori_loop(0, nch_my, body, 0)
    plsc.subcore_barrier()
    pltpu.sync_copy(shared.at[pl.ds(r0, RPT)],
                    out_hbm.at[c, pl.ds(r0, RPT)])

  return prop


def _make_prop(width, nbuf, a_chunks):
  ring = 2 * nbuf
  b_chunks = 2 * NCH - a_chunks

  if nbuf == 1:
    return _make_prop_simple(width, a_chunks, b_chunks)
  assert a_chunks % ring == 0 and b_chunks % ring == 0

  @functools.partial(
      pl.kernel,
      out_type=jax.ShapeDtypeStruct((2, NPAD, width), jnp.float32),
      mesh=_sc_mesh,
      compiler_params=_sc_params,
      scratch_types=(
          [pltpu.VMEM((ring, 2, CHUNK), jnp.int32)]
          + [pltpu.VMEM((CHUNK, width), jnp.float32)] * nbuf
          + [pltpu.VMEM_SHARED((NPAD, width), jnp.float32)]
          + [pltpu.SemaphoreType.DMA] * (ring + 2 * nbuf)
      ),
  )
  def prop(xp_hbm, edges_hbm, zeros_hbm, out_hbm, iring, *rest):
    rows = rest[:nbuf]
    shared = rest[nbuf]
    isem = rest[nbuf + 1:nbuf + 1 + ring]
    gsem = rest[nbuf + 1 + ring:nbuf + 1 + ring + nbuf]
    ssem = rest[nbuf + 1 + ring + nbuf:]
    c = lax.axis_index("c")
    s = lax.axis_index("s")
    r0 = pl.multiple_of(s * RPT, 8)
    base = jnp.where(c == 0, s * a_chunks, 16 * a_chunks + s * b_chunks)
    nch_my = jnp.where(c == 0, a_chunks, b_chunks)
    pltpu.sync_copy(zeros_hbm.at[pl.ds(r0, RPT)], shared.at[pl.ds(r0, RPT)])

    def fire_idx(k, r):
      pltpu.async_copy(edges_hbm.at[base + k], iring.at[r], isem[r])

    def wait_idx(k, r):
      pltpu.make_async_copy(edges_hbm.at[base + k], iring.at[r],
                            isem[r]).wait()

    def fire_gather(r, b):
      pltpu.async_copy(xp_hbm.at[iring.at[r, 0]], rows[b], gsem[b])

    plsc.subcore_barrier()
    for r in range(ring):
      fire_idx(r, r)
    for b in range(nbuf):
      wait_idx(b, b)
      fire_gather(b, b)

    def body(i, carry):
      K = i * ring
      for g in range(ring // nbuf):
        for b in range(nbuf):
          p = g * nbuf + b
          pltpu.make_async_copy(xp_hbm.at[iring.at[p, 0]], rows[b],
                                gsem[b]).wait()
          pltpu.async_copy(rows[b], shared.at[iring.at[p, 1]], ssem[b],
                           add=True)
        for b in range(nbuf):
          p = g * nbuf + b
          k = K + p
          pltpu.make_async_copy(rows[b], shared.at[iring.at[p, 1]],
                                ssem[b]).wait()

          @pl.when(k + ring < nch_my)
          def _():
            fire_idx(k + ring, p)

          @pl.when(k + nbuf < nch_my)
          def _():
            wait_idx(k + nbuf, (p + nbuf) % ring)
            fire_gather((p + nbuf) % ring, b)

      return carry

    lax.fori_loop(0, nch_my // ring, body, 0)
    plsc.subcore_barrier()
    pltpu.sync_copy(shared.at[pl.ds(r0, RPT)],
                    out_hbm.at[c, pl.ds(r0, RPT)])

  return prop


def _make_prop_pipe_staged(width, nbuf, chunk):
  nch = EPT // chunk
  assert nch * chunk == EPT and nch % nbuf == 0

  @functools.partial(
      pl.kernel,
      out_type=jax.ShapeDtypeStruct((2, NPAD, width), jnp.float32),
      mesh=_sc_mesh,
      compiler_params=_sc_params,
      scratch_types=(
          [pltpu.VMEM((nch, 2, chunk), jnp.int32)]
          + [pltpu.VMEM((chunk, width), jnp.float32)] * nbuf
          + [pltpu.VMEM_SHARED((NPAD, width), jnp.float32)]
          + [pltpu.SemaphoreType.DMA] * (2 * nbuf)
      ),
  )
  def prop(xp_hbm, edges_hbm, zeros_hbm, out_hbm, idx, *rest):
    rows = rest[:nbuf]
    shared = rest[nbuf]
    gsem = rest[nbuf + 1:nbuf + 1 + nbuf]
    ssem = rest[nbuf + 1 + nbuf:]
    c = lax.axis_index("c")
    s = lax.axis_index("s")
    wid = c * 16 + s
    r0 = pl.multiple_of(s * RPT, 8)
    pltpu.sync_copy(zeros_hbm.at[pl.ds(r0, RPT)], shared.at[pl.ds(r0, RPT)])
    pltpu.sync_copy(edges_hbm.at[wid], idx)
    plsc.subcore_barrier()

    for b in range(nbuf):
      pltpu.async_copy(xp_hbm.at[idx.at[b, 0]], rows[b], gsem[b])

    def body(j, carry):
      k0 = j * nbuf
      for b in range(nbuf):
        k = k0 + b
        pltpu.make_async_copy(xp_hbm.at[idx.at[k, 0]], rows[b],
                              gsem[b]).wait()
        pltpu.async_copy(rows[b], shared.at[idx.at[k, 1]], ssem[b],
                         add=True)
      for b in range(nbuf):
        k = k0 + b
        pltpu.make_async_copy(rows[b], shared.at[idx.at[k, 1]],
                              ssem[b]).wait()

        @pl.when(k + nbuf < nch)
        def _():
          pltpu.async_copy(xp_hbm.at[idx.at[k + nbuf, 0]], rows[b], gsem[b])

      return carry

    lax.fori_loop(0, nch // nbuf, body, 0)
    plsc.subcore_barrier()
    pltpu.sync_copy(shared.at[pl.ds(r0, RPT)],
                    out_hbm.at[c, pl.ds(r0, RPT)])

  return prop


def _make_prop128_colsplit(nbuf):
  ring = 2 * nbuf
  hw = D // 2
  tch = 2 * NCH
  assert tch % ring == 0

  @functools.partial(
      pl.kernel,
      out_type=jax.ShapeDtypeStruct((2, NPAD, hw), jnp.float32),
      mesh=_sc_mesh,
      compiler_params=_sc_params,
      scratch_types=(
          [pltpu.VMEM((ring, 2, CHUNK), jnp.int32)]
          + [pltpu.VMEM((CHUNK, hw), jnp.float32)] * nbuf
          + [pltpu.VMEM_SHARED((NPAD, hw), jnp.float32)] * 2
          + [pltpu.SemaphoreType.DMA] * (ring + 2 * nbuf)
      ),
  )
  def prop(xp_hbm, src_hbm, dst_hbm, zeros_hbm, out_hbm, iring, *rest):
    rows = rest[:nbuf]
    xp_sh = rest[nbuf]
    acc_sh = rest[nbuf + 1]
    isem = rest[nbuf + 2:nbuf + 2 + ring]
    gsem = rest[nbuf + 2 + ring:nbuf + 2 + ring + nbuf]
    ssem = rest[nbuf + 2 + ring + nbuf:]
    c = lax.axis_index("c")
    s = lax.axis_index("s")
    r0 = pl.multiple_of(s * RPT, 8)
    col = pl.multiple_of(c * hw, 8)
    base = s * tch
    pltpu.sync_copy(zeros_hbm, acc_sh.at[pl.ds(r0, RPT)])
    pltpu.sync_copy(xp_hbm.at[pl.ds(r0, RPT), pl.ds(col, hw)],
                    xp_sh.at[pl.ds(r0, RPT)])

    def fire_idx(k, r):
      pltpu.async_copy(src_hbm.at[base + k], iring.at[r, 0], isem[r])
      pltpu.async_copy(dst_hbm.at[base + k], iring.at[r, 1], isem[r])

    def wait_idx(k, r):
      pltpu.make_async_copy(src_hbm.at[base + k], iring.at[r, 0],
                            isem[r]).wait()
      pltpu.make_async_copy(dst_hbm.at[base + k], iring.at[r, 1],
                            isem[r]).wait()

    def fire_gather(r, b):
      pltpu.async_copy(xp_sh.at[iring.at[r, 0]], rows[b], gsem[b])

    plsc.subcore_barrier()
    for r in range(ring):
      fire_idx(r, r)
    for b in range(nbuf):
      wait_idx(b, b)
      fire_gather(b, b)

    def body(i, carry):
      K = i * ring
      for g in range(ring // nbuf):
        for b in range(nbuf):
          p = g * nbuf + b
          pltpu.make_async_copy(xp_sh.at[iring.at[p, 0]], rows[b],
                                gsem[b]).wait()
          pltpu.async_copy(rows[b], acc_sh.at[iring.at[p, 1]], ssem[b],
                           add=True)
        for b in range(nbuf):
          p = g * nbuf + b
          k = K + p
          pltpu.make_async_copy(rows[b], acc_sh.at[iring.at[p, 1]],
                                ssem[b]).wait()

          @pl.when(k + ring < tch)
          def _():
            fire_idx(k + ring, p)

          @pl.when(k + nbuf < tch)
          def _():
            wait_idx(k + nbuf, (p + nbuf) % ring)
            fire_gather((p + nbuf) % ring, b)

      return carry

    lax.fori_loop(0, tch // ring, body, 0)
    plsc.subcore_barrier()
    pltpu.sync_copy(acc_sh.at[pl.ds(r0, RPT)],
                    out_hbm.at[c, pl.ds(r0, RPT)])

  return prop


def _make_prop16_spmem(nbuf):
  @functools.partial(
      pl.kernel,
      out_type=jax.ShapeDtypeStruct((2, NPAD, DL), jnp.float32),
      mesh=_sc_mesh,
      compiler_params=_sc_params,
      scratch_types=(
          [pltpu.VMEM((NCH, CHUNK), jnp.int32)] * 2
          + [pltpu.VMEM((CHUNK, DL), jnp.float32)] * nbuf
          + [pltpu.VMEM_SHARED((NPAD, DL), jnp.float32)] * 2
          + [pltpu.SemaphoreType.DMA] * (2 * nbuf)
      ),
  )
  def prop(xp_hbm, src_hbm, dst_hbm, zeros_hbm, out_hbm, sbuf, dbuf, *rest):
    rows = rest[:nbuf]
    xp_sh = rest[nbuf]
    acc_sh = rest[nbuf + 1]
    gsem = rest[nbuf + 2:nbuf + 2 + nbuf]
    ssem = rest[nbuf + 2 + nbuf:]
    c = lax.axis_index("c")
    s = lax.axis_index("s")
    wid = c * 16 + s
    r0 = pl.multiple_of(s * RPT, 8)
    pltpu.sync_copy(zeros_hbm, acc_sh.at[pl.ds(r0, RPT)])
    pltpu.sync_copy(xp_hbm.at[pl.ds(r0, RPT)], xp_sh.at[pl.ds(r0, RPT)])
    pltpu.sync_copy(src_hbm.at[pl.ds(wid * NCH, NCH)], sbuf)
    pltpu.sync_copy(dst_hbm.at[pl.ds(wid * NCH, NCH)], dbuf)
    plsc.subcore_barrier()

    for b in range(nbuf):
      pltpu.async_copy(xp_sh.at[sbuf.at[b]], rows[b], gsem[b])

    def body(j, carry):
      k0 = j * nbuf
      for b in range(nbuf):
        k = k0 + b
        pltpu.make_async_copy(xp_sh.at[sbuf.at[k]], rows[b],
                              gsem[b]).wait()
        pltpu.async_copy(rows[b], acc_sh.at[dbuf.at[k]], ssem[b],
                         add=True)
      for b in range(nbuf):
        k = k0 + b
        pltpu.make_async_copy(rows[b], acc_sh.at[dbuf.at[k]],
                              ssem[b]).wait()

        @pl.when(k + nbuf < NCH)
        def _():
          pltpu.async_copy(xp_sh.at[sbuf.at[k + nbuf]], rows[b], gsem[b])

      return carry

    lax.fori_loop(0, NCH // nbuf, body, 0)
    plsc.subcore_barrier()
    pltpu.sync_copy(acc_sh.at[pl.ds(r0, RPT)],
                    out_hbm.at[c, pl.ds(r0, RPT)])

  return prop


_prop128 = _make_prop128_colsplit(2)
_prop16 = _make_prop16_spmem(2)


@functools.partial(
    pl.kernel,
    out_type=jax.ShapeDtypeStruct((2, NPAD, DL), jnp.float32),
    mesh=_sc_mesh,
    compiler_params=_sc_params,
    scratch_types=(
        [pltpu.VMEM((NCH, CHUNK), jnp.int32),
         pltpu.VMEM((CHUNK, DL), jnp.float32),
         pltpu.VMEM_SHARED((NPAD, DL), jnp.float32)]
        + [pltpu.SemaphoreType.DMA] * 8
    ),
)
def _deg_kernel(dst_hbm, zeros_hbm, ones_hbm, out_hbm,
                dstbuf, onesv, shared, *sems):
  c = lax.axis_index("c")
  s = lax.axis_index("s")
  wid = c * 16 + s
  r0 = pl.multiple_of(s * RPT, 8)
  pltpu.sync_copy(zeros_hbm, shared.at[pl.ds(r0, RPT)])
  pltpu.sync_copy(ones_hbm, onesv)
  pltpu.sync_copy(dst_hbm.at[pl.ds(wid * NCH, NCH)], dstbuf)
  plsc.subcore_barrier()

  def body(j, carry):
    for b in range(8):
      pltpu.async_copy(onesv, shared.at[dstbuf.at[j * 8 + b]], sems[b],
                       add=True)
    for b in range(8):
      pltpu.make_async_copy(onesv, shared.at[dstbuf.at[j * 8 + b]],
                            sems[b]).wait()
    return carry

  lax.fori_loop(0, NCH // 8, body, 0)
  plsc.subcore_barrier()
  pltpu.sync_copy(shared.at[pl.ds(r0, RPT)],
                  out_hbm.at[c, pl.ds(r0, RPT)])


@functools.partial(
    pl.kernel,
    out_type=jax.ShapeDtypeStruct((NIDPAD, 2 * DL), jnp.float32),
    mesh=_sc_mesh,
    compiler_params=_sc_params,
    scratch_types=[
        pltpu.VMEM((IDS_PT,), jnp.int32),
        pltpu.VMEM((IDS_PT, 2 * DL), jnp.float32),
        pltpu.SemaphoreType.DMA,
    ],
)
def _gather_kernel(comb_hbm, ids_hbm, out_hbm, idbuf, rowsbuf, sem):
  c = lax.axis_index("c")
  s = lax.axis_index("s")
  wid = c * 16 + s
  pltpu.sync_copy(ids_hbm.at[wid], idbuf)
  pltpu.async_copy(comb_hbm.at[idbuf], rowsbuf, sem).wait()
  pltpu.sync_copy(rowsbuf, out_hbm.at[pl.ds(pl.multiple_of(wid * IDS_PT, 8),
                                            IDS_PT)])


BR = 640


def _mm1_body(x_ref, w_ref, degp_ref, xp_ref, dinv_ref):
  deg = degp_ref[0] + degp_ref[1] + 1.0
  dinv = lax.rsqrt(jnp.maximum(deg, 1.0))
  h = jnp.dot(x_ref[...], w_ref[...], preferred_element_type=jnp.float32)
  xp_ref[...] = h * dinv[:, :1]
  dinv_ref[...] = dinv


_mm1 = pl.pallas_call(
    _mm1_body,
    grid=(NPAD // BR,),
    in_specs=[
        pl.BlockSpec((BR, D), lambda i: (i, 0)),
        pl.BlockSpec((D, D), lambda i: (0, 0)),
        pl.BlockSpec((2, BR, DL), lambda i: (0, i, 0)),
    ],
    out_specs=[
        pl.BlockSpec((BR, D), lambda i: (i, 0)),
        pl.BlockSpec((BR, DL), lambda i: (i, 0)),
    ],
    out_shape=[
        jax.ShapeDtypeStruct((NPAD, D), jnp.float32),
        jax.ShapeDtypeStruct((NPAD, DL), jnp.float32),
    ],
)


def _mid_body(s1p_ref, xp1_ref, dinv_ref, b1_ref, w2_ref, xp2_ref):
  dinv = dinv_ref[...][:, :1]
  s1 = jnp.concatenate([s1p_ref[0], s1p_ref[1]], axis=1)
  out1 = (s1 + xp1_ref[...]) * dinv + b1_ref[...]
  h1 = jnp.maximum(out1, 0.0)
  xp2_ref[...] = jnp.dot(h1, w2_ref[...],
                         preferred_element_type=jnp.float32) * dinv


_mid = pl.pallas_call(
    _mid_body,
    grid=(NPAD // BR,),
    in_specs=[
        pl.BlockSpec((2, BR, D // 2), lambda i: (0, i, 0)),
        pl.BlockSpec((BR, D), lambda i: (i, 0)),
        pl.BlockSpec((BR, DL), lambda i: (i, 0)),
        pl.BlockSpec((1, D), lambda i: (0, 0)),
        pl.BlockSpec((D, DL), lambda i: (0, 0)),
    ],
    out_specs=pl.BlockSpec((BR, DL), lambda i: (i, 0)),
    out_shape=jax.ShapeDtypeStruct((NPAD, DL), jnp.float32),
)


def _pack_body(s2p_ref, xp2_ref, dinv_ref, b2_ref, lab_ref, comb_ref):
  dinv = dinv_ref[...][:, :1]
  out2 = (s2p_ref[0] + s2p_ref[1] + xp2_ref[...]) * dinv + b2_ref[...]
  lab = jnp.broadcast_to(lab_ref[...], (BR, DL))
  labf = lax.bitcast_convert_type(lab, jnp.float32)
  comb_ref[...] = jnp.concatenate([out2, labf], axis=1)


_pack = pl.pallas_call(
    _pack_body,
    grid=(NPAD // BR,),
    in_specs=[
        pl.BlockSpec((2, BR, DL), lambda i: (0, i, 0)),
        pl.BlockSpec((BR, DL), lambda i: (i, 0)),
        pl.BlockSpec((BR, DL), lambda i: (i, 0)),
        pl.BlockSpec((1, DL), lambda i: (0, 0)),
        pl.BlockSpec((BR, 1), lambda i: (i, 0)),
    ],
    out_specs=pl.BlockSpec((BR, 2 * DL), lambda i: (i, 0)),
    out_shape=jax.ShapeDtypeStruct((NPAD, 2 * DL), jnp.float32),
)


def _head_body(comb_ref, loss_ref):
  z = comb_ref[...]
  logits = z[:, :DL]
  lab = lax.bitcast_convert_type(z[:, DL:], jnp.int32)
  col = lax.broadcasted_iota(jnp.int32, (NIDPAD, DL), 1)
  picked = jnp.sum(jnp.where(col == lab, logits, 0.0), axis=1, keepdims=True)
  m = jnp.max(logits, axis=1, keepdims=True)
  lse = jnp.log(jnp.sum(jnp.exp(logits - m), axis=1, keepdims=True)) + m
  rowi = lax.broadcasted_iota(jnp.int32, (NIDPAD, 1), 0)
  mask = jnp.where(rowi < NID, 1.0, 0.0)
  loss_ref[...] = jnp.reshape(jnp.sum((lse - picked) * mask) / NID, (1, 1))


_head = pl.pallas_call(
    _head_body,
    grid=(1,),
    in_specs=[pl.BlockSpec((NIDPAD, 2 * DL), lambda i: (0, 0))],
    out_specs=pl.BlockSpec((1, 1), lambda i: (0, 0)),
    out_shape=jax.ShapeDtypeStruct((1, 1), jnp.float32),
)


def kernel(x, edge_index, node_ids, label_inds, W1, b1, W2, b2):
  x_pad = jnp.concatenate([x, jnp.zeros((NPAD - N, D), jnp.float32)], axis=0)
  pad_src = jnp.zeros((EPAD - E,), jnp.int32)
  pad_dst = (N + (jnp.arange(EPAD - E, dtype=jnp.int32) % (NPAD - N)))
  src2d = jnp.concatenate([edge_index[0], pad_src]).reshape(-1, CHUNK)
  dst2d = jnp.concatenate([edge_index[1], pad_dst]).reshape(-1, CHUNK)
  ids = jnp.concatenate(
      [node_ids.astype(jnp.int32),
       jnp.zeros((NIDPAD - NID,), jnp.int32)]).reshape(NW, IDS_PT)
  lab1 = jnp.concatenate(
      [label_inds.astype(jnp.int32),
       jnp.zeros((NPAD - N,), jnp.int32)]).reshape(NPAD, 1)
  zecol = jnp.zeros((RPT, D // 2), jnp.float32)
  ze16 = jnp.zeros((RPT, DL), jnp.float32)
  ones16 = jnp.ones((CHUNK, DL), jnp.float32)

  degp = _deg_kernel(dst2d, ze16, ones16)
  xp1, dinv16 = _mm1(x_pad, W1, degp)
  s1p = _prop128(xp1, src2d, dst2d, zecol)
  xp2 = _mid(s1p, xp1, dinv16, b1.reshape(1, D), W2)
  s2p = _prop16(xp2, src2d, dst2d, ze16)
  comb = _pack(s2p, xp2, dinv16, b2.reshape(1, DL), lab1)
  g = _gather_kernel(comb, ids)
  lossm = _head(g)
  return (lossm[0, 0], g[:NID, :DL])
